# baseline (speedup 1.0000x reference)
import functools

import jax
import jax.numpy as jnp
from jax import lax
from jax.experimental import pallas as pl
from jax.experimental.pallas import tpu as pltpu
from jax.experimental.pallas import tpu_sc as plsc

N = 10000
E = 320000
D = 128
H = 128
T = 3

NC = 2
NS = 16
NW = NC * NS
EPW = E // NW
CHUNK = 112
PAD = -EPW % CHUNK
EPWP = EPW + PAD
NCH = EPWP // CHUNK
DCH = 80
DNCH = EPW // DCH
RPT = N // NS
RPT8 = 624

_mesh = plsc.VectorSubcoreMesh(core_axis_name="c", subcore_axis_name="s")


def _worker(c, s):
    return s * NC + c


@functools.partial(
    pl.kernel,
    out_type=jax.ShapeDtypeStruct((NC * N,), jnp.float32),
    mesh=_mesh,
    scratch_types=[
        pltpu.VMEM((DNCH, DCH), jnp.int32),
        pltpu.VMEM((DCH,), jnp.float32),
        pltpu.VMEM((RPT8,), jnp.float32),
        pltpu.VMEM_SHARED((N,), jnp.float32),
        pltpu.SemaphoreType.DMA,
        pltpu.SemaphoreType.DMA,
    ],
)
def _degree_kernel(col3_hbm, out_hbm, col_v2, ones_v, zeros_v, acc_sh,
                   ssem0, ssem1):
    c = lax.axis_index("c")
    s = lax.axis_index("s")
    wid = _worker(c, s)
    ssems = (ssem0, ssem1)

    pltpu.sync_copy(col3_hbm.at[wid], col_v2)

    def fill(i, _):
        ones_v[pl.ds(i * 16, 16)] = jnp.ones((16,), jnp.float32)
        return _

    lax.fori_loop(0, DCH // 16, fill, 0)

    def zfill(i, _):
        zeros_v[pl.ds(i * 16, 16)] = jnp.zeros((16,), jnp.float32)
        return _

    lax.fori_loop(0, RPT8 // 16, zfill, 0)

    pltpu.sync_copy(zeros_v, acc_sh.at[pl.ds(s * RPT8, RPT8)])

    @pl.when(s == NS - 1)
    def _():
        pltpu.sync_copy(zeros_v.at[pl.ds(0, 16)],
                        acc_sh.at[pl.ds(NS * RPT8, N - NS * RPT8)])

    plsc.subcore_barrier()

    def s_start(k, j):
        pltpu.async_copy(ones_v, acc_sh.at[col_v2.at[k]], ssems[j], add=True)

    def s_wait(j):
        pltpu.make_async_copy(ones_v, acc_sh.at[col_v2.at[0]],
                              ssems[j]).wait()

    def stage(k, j):
        @pl.when(k >= 2)
        def _():
            s_wait(j)

        s_start(k, j)

    def pair(i, _):
        stage(2 * i, 0)
        stage(2 * i + 1, 1)
        return _

    lax.fori_loop(0, DNCH // 2, pair, 0)
    stage(DNCH - 1, 0)
    s_wait(1)
    s_wait(0)
    plsc.subcore_barrier()

    obase = c * N
    pltpu.sync_copy(acc_sh.at[pl.ds(s * RPT8, RPT8)], zeros_v)
    pltpu.sync_copy(zeros_v, out_hbm.at[pl.ds(obase + s * RPT8, RPT8)])

    @pl.when(s == NS - 1)
    def _():
        tail = N - NS * RPT8
        pltpu.sync_copy(acc_sh.at[pl.ds(NS * RPT8, tail)],
                        ones_v.at[pl.ds(0, tail)])
        pltpu.sync_copy(ones_v.at[pl.ds(0, tail)],
                        out_hbm.at[pl.ds(obase + NS * RPT8, tail)])


_part_out = jax.ShapeDtypeStruct((NC, N, D), jnp.float32)


@functools.partial(
    pl.kernel,
    out_type=_part_out,
    mesh=_mesh,
    scratch_types=[
        pltpu.VMEM((EPWP,), jnp.int32),
        pltpu.VMEM((EPWP,), jnp.int32),
        pltpu.VMEM((CHUNK, D), jnp.float32),
        pltpu.VMEM((CHUNK, D), jnp.float32),
        pltpu.VMEM_SHARED((N + NS, D), jnp.float32),
        pltpu.SemaphoreType.DMA,
        pltpu.SemaphoreType.DMA,
        pltpu.SemaphoreType.DMA,
    ],
)
def _spmv_kernel(v_hbm, row_hbm, col_hbm, out_hbm,
                 row_f, col_f, buf0, buf1, acc_sh, gsem0, gsem1, ssem):
    c = lax.axis_index("c")
    s = lax.axis_index("s")
    wid = _worker(c, s)
    bufs = (buf0, buf1)
    gsems = (gsem0, gsem1)

    ebase = wid * EPWP
    pltpu.sync_copy(row_hbm.at[pl.ds(ebase, EPWP)], row_f)
    pltpu.sync_copy(col_hbm.at[pl.ds(ebase, EPWP)], col_f)

    def zrow(i, _):
        for j in range(D // 16):
            buf0[i, pl.ds(j * 16, 16)] = jnp.zeros((16,), jnp.float32)
        return _

    lax.fori_loop(0, CHUNK, zrow, 0)

    rbase = s * RPT8
    rem = RPT8 % CHUNK
    tail = N - NS * RPT8

    def zcp(i, _):
        pltpu.sync_copy(buf0, acc_sh.at[pl.ds(rbase + i * CHUNK, CHUNK)])
        return _

    lax.fori_loop(0, RPT8 // CHUNK, zcp, 0)
    pltpu.sync_copy(buf0.at[pl.ds(0, rem)],
                    acc_sh.at[pl.ds(rbase + RPT8 - rem, rem)])

    @pl.when(s == NS - 1)
    def _():
        pltpu.sync_copy(buf0.at[pl.ds(0, tail)],
                        acc_sh.at[pl.ds(NS * RPT8, tail)])

    plsc.subcore_barrier()

    def g_start(k, j):
        pltpu.async_copy(v_hbm.at[row_f.at[pl.ds(k * CHUNK, CHUNK)]],
                         bufs[j], gsems[j])

    def g_wait(j):
        pltpu.make_async_copy(v_hbm.at[row_f.at[pl.ds(0, CHUNK)]],
                              bufs[j], gsems[j]).wait()

    def s_start(k, j):
        pltpu.async_copy(bufs[j],
                         acc_sh.at[col_f.at[pl.ds(k * CHUNK, CHUNK)]],
                         ssem, add=True)

    def s_wait(j):
        pltpu.make_async_copy(bufs[j],
                              acc_sh.at[col_f.at[pl.ds(0, CHUNK)]],
                              ssem).wait()

    def stage(k, q):
        j = q % 2
        j2 = 1 - j

        @pl.when(k >= 1)
        def _():
            s_wait(j2)

        @pl.when(k + 1 < NCH)
        def _():
            g_start(k + 1, j2)

        g_wait(j)
        s_start(k, j)

    g_start(0, 0)

    def pair(i, _):
        stage(2 * i, 0)
        stage(2 * i + 1, 1)
        return _

    lax.fori_loop(0, NCH // 2, pair, 0)
    s_wait((NCH - 1) % 2)
    plsc.subcore_barrier()

    def wcp(i, _):
        pltpu.sync_copy(acc_sh.at[pl.ds(rbase + i * CHUNK, CHUNK)], buf0)
        pltpu.sync_copy(buf0,
                        out_hbm.at[c, pl.ds(rbase + i * CHUNK, CHUNK)])
        return _

    lax.fori_loop(0, RPT8 // CHUNK, wcp, 0)
    pltpu.sync_copy(acc_sh.at[pl.ds(rbase + RPT8 - rem, rem)],
                    buf0.at[pl.ds(0, rem)])
    pltpu.sync_copy(buf0.at[pl.ds(0, rem)],
                    out_hbm.at[c, pl.ds(rbase + RPT8 - rem, rem)])

    @pl.when(s == NS - 1)
    def _():
        pltpu.sync_copy(acc_sh.at[pl.ds(NS * RPT8, tail)],
                        buf1.at[pl.ds(0, tail)])
        pltpu.sync_copy(buf1.at[pl.ds(0, tail)],
                        out_hbm.at[c, pl.ds(NS * RPT8, tail)])


_BR = 1000
_GRID = N // _BR


def _tc_a_body(x_ref, w0_ref, degp_ref, u0_ref, ut0_ref, ut1_ref, ut2_ref):
    u = jnp.dot(x_ref[...], w0_ref[...], preferred_element_type=jnp.float32)
    u0_ref[...] = u
    deg = degp_ref[0] + degp_ref[1]
    ut_refs = (ut0_ref, ut1_ref, ut2_ref)
    for t in range(T):
        ct = 0.5 ** t
        d = lax.rsqrt(ct * deg + 1.0)
        ut_refs[t][...] = d * u


def _tc_b_body(ct_ref, p_ref, u0_ref, degp_ref, w1_ref, b0_ref,
               a_ref, v_ref, u1_ref):
    u0 = u0_ref[...]
    deg = degp_ref[0] + degp_ref[1]
    ct = ct_ref[0]
    d = lax.rsqrt(ct * deg + 1.0)
    st = p_ref[0] + p_ref[1]
    z = d * (ct * st) + (d * d) * u0 + b0_ref[...]
    z = jnp.where(z >= 0, z, a_ref[...] * z)
    u1 = jnp.dot(z, w1_ref[...], preferred_element_type=jnp.float32)
    u1_ref[...] = u1
    v_ref[...] = d * u1


def _tc_c_body(p0_ref, p1_ref, p2_ref, u10_ref, u11_ref, u12_ref, degp_ref,
               b1_ref, a_ref, mix_ref, out_ref):
    deg = degp_ref[0] + degp_ref[1]
    b1 = b1_ref[...]
    a = a_ref[...]
    m = mix_ref[...]
    e = jnp.exp(m - jnp.max(m, axis=0, keepdims=True))
    coeff = e / jnp.sum(e, axis=0, keepdims=True)
    p_refs = (p0_ref, p1_ref, p2_ref)
    u1_refs = (u10_ref, u11_ref, u12_ref)
    acc = jnp.zeros((_BR, D), jnp.float32)
    for t in range(T):
        ct = 0.5 ** t
        d = lax.rsqrt(ct * deg + 1.0)
        st = p_refs[t][0] + p_refs[t][1]
        u1 = u1_refs[t][...]
        z = d * (ct * st) + (d * d) * u1 + b1
        z = jnp.where(z >= 0, z, a * z)
        acc = acc + coeff[t:t + 1, 0:1] * z
    out_ref[...] = acc


def _row_block(i):
    return (i, 0)


_spec_rows = pl.BlockSpec((_BR, D), _row_block)
_spec_full = pl.BlockSpec((D, D), lambda i: (0, 0))
_spec_vec = pl.BlockSpec((1, D), lambda i: (0, 0))
_spec_deg = pl.BlockSpec((NC, _BR, 1), lambda i: (0, i, 0))
_spec_part = pl.BlockSpec((NC, _BR, D), lambda i: (0, i, 0))
_spec_mix = pl.BlockSpec((8, 1), lambda i: (0, 0))

_rows_out = jax.ShapeDtypeStruct((N, D), jnp.float32)

_tc_a = pl.pallas_call(
    _tc_a_body,
    grid=(_GRID,),
    in_specs=[_spec_rows, _spec_full, _spec_deg],
    out_specs=[_spec_rows] * 4,
    out_shape=[_rows_out] * 4,
)

_spec_ct = pl.BlockSpec(memory_space=pltpu.SMEM)

_tc_b = pl.pallas_call(
    _tc_b_body,
    grid=(_GRID,),
    in_specs=[_spec_ct, _spec_part, _spec_rows, _spec_deg,
              _spec_full, _spec_vec, _spec_vec],
    out_specs=[_spec_rows] * 2,
    out_shape=[_rows_out] * 2,
)

_tc_c = pl.pallas_call(
    _tc_c_body,
    grid=(_GRID,),
    in_specs=[_spec_part, _spec_part, _spec_part, _spec_rows, _spec_rows,
              _spec_rows, _spec_deg, _spec_vec, _spec_vec, _spec_mix],
    out_specs=_spec_rows,
    out_shape=_rows_out,
)


def kernel(x, edge_index, W0, b0, W1, b1, prelu_a, mixing):
    row = edge_index[0]
    col = edge_index[1]
    b0r = b0.reshape(1, H)
    b1r = b1.reshape(1, H)
    ar = prelu_a.reshape(1, H)
    mixp = jnp.pad(mixing.astype(jnp.float32), ((0, 8 - T), (0, 0)),
                   constant_values=-1e30)

    col3 = col.reshape(NW, DNCH, DCH)
    w = jnp.arange(NW, dtype=jnp.int32)[:, None]
    prow = (w * 113 + jnp.arange(PAD, dtype=jnp.int32)[None, :] * 89) % N
    pcol = jnp.full((NW, PAD), N, dtype=jnp.int32) + w // NC
    rowp = jnp.concatenate([row.reshape(NW, EPW), prow], axis=1).reshape(-1)
    colp = jnp.concatenate([col.reshape(NW, EPW), pcol], axis=1).reshape(-1)

    degp = _degree_kernel(col3).reshape(NC, N, 1)

    u0, ut0, ut1, ut2 = _tc_a(x, W0, degp)

    cts = [jnp.full((1,), 0.5 ** t, dtype=jnp.float32) for t in range(T)]

    p10 = _spmv_kernel(ut0, rowp, colp)
    p11 = _spmv_kernel(ut1, rowp, colp)
    v0, u10 = _tc_b(cts[0], p10, u0, degp, W1, b0r, ar)
    p12 = _spmv_kernel(ut2, rowp, colp)
    v1, u11 = _tc_b(cts[1], p11, u0, degp, W1, b0r, ar)
    p20 = _spmv_kernel(v0, rowp, colp)
    v2, u12 = _tc_b(cts[2], p12, u0, degp, W1, b0r, ar)
    p21 = _spmv_kernel(v1, rowp, colp)
    p22 = _spmv_kernel(v2, rowp, colp)

    features = _tc_c(p20, p21, p22, u10, u11, u12, degp, b1r, ar, mixp)

    edge_weight_last = jnp.full((E,), 0.25, dtype=jnp.float32)
    return (features, edge_index, edge_weight_last)


# speedup vs baseline: 1.0228x; 1.0228x over previous
import functools

import jax
import jax.numpy as jnp
from jax import lax
from jax.experimental import pallas as pl
from jax.experimental.pallas import tpu as pltpu
from jax.experimental.pallas import tpu_sc as plsc

N = 10000
E = 320000
D = 128
H = 128
T = 3

NC = 2
NS = 16
NW = NC * NS
EPW = E // NW
CHUNK = 128
PAD = -EPW % CHUNK
EPWP = EPW + PAD
NCH = EPWP // CHUNK
DCH = 80
DNCH = EPW // DCH
RPT = N // NS
RPT8 = 624

_mesh = plsc.VectorSubcoreMesh(core_axis_name="c", subcore_axis_name="s")


def _worker(c, s):
    return s * NC + c


@functools.partial(
    pl.kernel,
    out_type=jax.ShapeDtypeStruct((NC * N,), jnp.float32),
    mesh=_mesh,
    scratch_types=[
        pltpu.VMEM((DNCH, DCH), jnp.int32),
        pltpu.VMEM((DCH,), jnp.float32),
        pltpu.VMEM((RPT8,), jnp.float32),
        pltpu.VMEM_SHARED((N,), jnp.float32),
        pltpu.SemaphoreType.DMA,
        pltpu.SemaphoreType.DMA,
    ],
)
def _degree_kernel(col3_hbm, out_hbm, col_v2, ones_v, zeros_v, acc_sh,
                   ssem0, ssem1):
    c = lax.axis_index("c")
    s = lax.axis_index("s")
    wid = _worker(c, s)
    ssems = (ssem0, ssem1)

    pltpu.sync_copy(col3_hbm.at[wid], col_v2)

    def fill(i, _):
        ones_v[pl.ds(i * 16, 16)] = jnp.ones((16,), jnp.float32)
        return _

    lax.fori_loop(0, DCH // 16, fill, 0)

    def zfill(i, _):
        zeros_v[pl.ds(i * 16, 16)] = jnp.zeros((16,), jnp.float32)
        return _

    lax.fori_loop(0, RPT8 // 16, zfill, 0)

    pltpu.sync_copy(zeros_v, acc_sh.at[pl.ds(s * RPT8, RPT8)])

    @pl.when(s == NS - 1)
    def _():
        pltpu.sync_copy(zeros_v.at[pl.ds(0, 16)],
                        acc_sh.at[pl.ds(NS * RPT8, N - NS * RPT8)])

    plsc.subcore_barrier()

    def s_start(k, j):
        pltpu.async_copy(ones_v, acc_sh.at[col_v2.at[k]], ssems[j], add=True)

    def s_wait(j):
        pltpu.make_async_copy(ones_v, acc_sh.at[col_v2.at[0]],
                              ssems[j]).wait()

    def stage(k, j):
        @pl.when(k >= 2)
        def _():
            s_wait(j)

        s_start(k, j)

    def pair(i, _):
        stage(2 * i, 0)
        stage(2 * i + 1, 1)
        return _

    lax.fori_loop(0, DNCH // 2, pair, 0)
    stage(DNCH - 1, 0)
    s_wait(1)
    s_wait(0)
    plsc.subcore_barrier()

    obase = c * N
    pltpu.sync_copy(acc_sh.at[pl.ds(s * RPT8, RPT8)], zeros_v)
    pltpu.sync_copy(zeros_v, out_hbm.at[pl.ds(obase + s * RPT8, RPT8)])

    @pl.when(s == NS - 1)
    def _():
        tail = N - NS * RPT8
        pltpu.sync_copy(acc_sh.at[pl.ds(NS * RPT8, tail)],
                        ones_v.at[pl.ds(0, tail)])
        pltpu.sync_copy(ones_v.at[pl.ds(0, tail)],
                        out_hbm.at[pl.ds(obase + NS * RPT8, tail)])


_part_out = jax.ShapeDtypeStruct((NC, N, D), jnp.float32)


@functools.partial(
    pl.kernel,
    out_type=_part_out,
    mesh=_mesh,
    scratch_types=[
        pltpu.VMEM((3, CHUNK), jnp.int32),
        pltpu.VMEM((3, CHUNK), jnp.int32),
        pltpu.VMEM((CHUNK, D), jnp.float32),
        pltpu.VMEM((CHUNK, D), jnp.float32),
        pltpu.VMEM((CHUNK, D), jnp.float32),
        pltpu.VMEM_SHARED((N + NS, D), jnp.float32),
        pltpu.SemaphoreType.DMA,
        pltpu.SemaphoreType.DMA,
        pltpu.SemaphoreType.DMA,
        pltpu.SemaphoreType.DMA,
        pltpu.SemaphoreType.DMA,
    ],
)
def _spmv_kernel(v_hbm, row_hbm, col_hbm, out_hbm,
                 row_i, col_i, buf0, buf1, buf2, acc_sh,
                 gsem0, gsem1, gsem2, ssem, isem):
    c = lax.axis_index("c")
    s = lax.axis_index("s")
    wid = _worker(c, s)
    bufs = (buf0, buf1, buf2)
    gsems = (gsem0, gsem1, gsem2)

    ebase = wid * EPWP

    def zrow(i, _):
        for j in range(D // 16):
            buf0[i, pl.ds(j * 16, 16)] = jnp.zeros((16,), jnp.float32)
        return _

    lax.fori_loop(0, CHUNK, zrow, 0)

    rbase = s * RPT8
    rem = RPT8 % CHUNK
    tail = N - NS * RPT8

    def zcp(i, _):
        pltpu.sync_copy(buf0, acc_sh.at[pl.ds(rbase + i * CHUNK, CHUNK)])
        return _

    lax.fori_loop(0, RPT8 // CHUNK, zcp, 0)
    pltpu.sync_copy(buf0.at[pl.ds(0, rem)],
                    acc_sh.at[pl.ds(rbase + RPT8 - rem, rem)])

    @pl.when(s == NS - 1)
    def _():
        pltpu.sync_copy(buf0.at[pl.ds(0, tail)],
                        acc_sh.at[pl.ds(NS * RPT8, tail)])

    plsc.subcore_barrier()

    def i_start(k, m):
        off = ebase + k * CHUNK
        pltpu.async_copy(row_hbm.at[pl.ds(off, CHUNK)], row_i.at[m], isem)
        pltpu.async_copy(col_hbm.at[pl.ds(off, CHUNK)], col_i.at[m], isem)

    def i_wait(m):
        pltpu.make_async_copy(row_hbm.at[pl.ds(0, CHUNK)], row_i.at[m],
                              isem).wait()
        pltpu.make_async_copy(col_hbm.at[pl.ds(0, CHUNK)], col_i.at[m],
                              isem).wait()

    def g_start(k, j):
        pltpu.async_copy(v_hbm.at[row_i.at[j]], bufs[j], gsems[j])

    def g_wait(j):
        pltpu.make_async_copy(v_hbm.at[row_i.at[0]], bufs[j],
                              gsems[j]).wait()

    def s_start(j):
        pltpu.async_copy(bufs[j], acc_sh.at[col_i.at[j]], ssem, add=True)

    def s_wait(j):
        pltpu.make_async_copy(bufs[j], acc_sh.at[col_i.at[0]], ssem).wait()

    def stage(k, q):
        j = q % 3

        @pl.when(k >= 1)
        def _():
            s_wait((q - 1) % 3)

        @pl.when(k + 1 < NCH)
        def _():
            i_wait((q + 1) % 3)
            g_start(k + 1, (q + 1) % 3)

        g_wait(j)
        s_start(j)

        @pl.when(k + 2 < NCH)
        def _():
            i_start(k + 2, (q + 2) % 3)

    i_start(0, 0)
    i_wait(0)
    g_start(0, 0)
    i_start(1, 1)

    def triple(i, _):
        for q in range(3):
            stage(3 * i + q, q)
        return _

    lax.fori_loop(0, NCH // 3, triple, 0)
    for k in range(NCH - NCH % 3, NCH):
        stage(k, k)
    s_wait((NCH - 1) % 3)
    plsc.subcore_barrier()

    def wcp(i, _):
        pltpu.sync_copy(acc_sh.at[pl.ds(rbase + i * CHUNK, CHUNK)], buf0)
        pltpu.sync_copy(buf0,
                        out_hbm.at[c, pl.ds(rbase + i * CHUNK, CHUNK)])
        return _

    lax.fori_loop(0, RPT8 // CHUNK, wcp, 0)
    pltpu.sync_copy(acc_sh.at[pl.ds(rbase + RPT8 - rem, rem)],
                    buf0.at[pl.ds(0, rem)])
    pltpu.sync_copy(buf0.at[pl.ds(0, rem)],
                    out_hbm.at[c, pl.ds(rbase + RPT8 - rem, rem)])

    @pl.when(s == NS - 1)
    def _():
        pltpu.sync_copy(acc_sh.at[pl.ds(NS * RPT8, tail)],
                        buf1.at[pl.ds(0, tail)])
        pltpu.sync_copy(buf1.at[pl.ds(0, tail)],
                        out_hbm.at[c, pl.ds(NS * RPT8, tail)])


_BR = 1000
_GRID = N // _BR


def _tc_a_body(x_ref, w0_ref, degp_ref, u0_ref, ut0_ref, ut1_ref, ut2_ref):
    u = jnp.dot(x_ref[...], w0_ref[...], preferred_element_type=jnp.float32)
    u0_ref[...] = u
    deg = degp_ref[0] + degp_ref[1]
    ut_refs = (ut0_ref, ut1_ref, ut2_ref)
    for t in range(T):
        ct = 0.5 ** t
        d = lax.rsqrt(ct * deg + 1.0)
        ut_refs[t][...] = d * u


def _tc_b_body(ct_ref, p_ref, u0_ref, degp_ref, w1_ref, b0_ref,
               a_ref, v_ref, u1_ref):
    u0 = u0_ref[...]
    deg = degp_ref[0] + degp_ref[1]
    ct = ct_ref[0]
    d = lax.rsqrt(ct * deg + 1.0)
    st = p_ref[0] + p_ref[1]
    z = d * (ct * st) + (d * d) * u0 + b0_ref[...]
    z = jnp.where(z >= 0, z, a_ref[...] * z)
    u1 = jnp.dot(z, w1_ref[...], preferred_element_type=jnp.float32)
    u1_ref[...] = u1
    v_ref[...] = d * u1


def _tc_c_body(p0_ref, p1_ref, p2_ref, u10_ref, u11_ref, u12_ref, degp_ref,
               b1_ref, a_ref, mix_ref, out_ref):
    deg = degp_ref[0] + degp_ref[1]
    b1 = b1_ref[...]
    a = a_ref[...]
    m = mix_ref[...]
    e = jnp.exp(m - jnp.max(m, axis=0, keepdims=True))
    coeff = e / jnp.sum(e, axis=0, keepdims=True)
    p_refs = (p0_ref, p1_ref, p2_ref)
    u1_refs = (u10_ref, u11_ref, u12_ref)
    acc = jnp.zeros((_BR, D), jnp.float32)
    for t in range(T):
        ct = 0.5 ** t
        d = lax.rsqrt(ct * deg + 1.0)
        st = p_refs[t][0] + p_refs[t][1]
        u1 = u1_refs[t][...]
        z = d * (ct * st) + (d * d) * u1 + b1
        z = jnp.where(z >= 0, z, a * z)
        acc = acc + coeff[t:t + 1, 0:1] * z
    out_ref[...] = acc


def _row_block(i):
    return (i, 0)


_spec_rows = pl.BlockSpec((_BR, D), _row_block)
_spec_full = pl.BlockSpec((D, D), lambda i: (0, 0))
_spec_vec = pl.BlockSpec((1, D), lambda i: (0, 0))
_spec_deg = pl.BlockSpec((NC, _BR, 1), lambda i: (0, i, 0))
_spec_part = pl.BlockSpec((NC, _BR, D), lambda i: (0, i, 0))
_spec_mix = pl.BlockSpec((8, 1), lambda i: (0, 0))

_rows_out = jax.ShapeDtypeStruct((N, D), jnp.float32)

_tc_a = pl.pallas_call(
    _tc_a_body,
    grid=(_GRID,),
    in_specs=[_spec_rows, _spec_full, _spec_deg],
    out_specs=[_spec_rows] * 4,
    out_shape=[_rows_out] * 4,
)

_spec_ct = pl.BlockSpec(memory_space=pltpu.SMEM)

_tc_b = pl.pallas_call(
    _tc_b_body,
    grid=(_GRID,),
    in_specs=[_spec_ct, _spec_part, _spec_rows, _spec_deg,
              _spec_full, _spec_vec, _spec_vec],
    out_specs=[_spec_rows] * 2,
    out_shape=[_rows_out] * 2,
)

_tc_c = pl.pallas_call(
    _tc_c_body,
    grid=(_GRID,),
    in_specs=[_spec_part, _spec_part, _spec_part, _spec_rows, _spec_rows,
              _spec_rows, _spec_deg, _spec_vec, _spec_vec, _spec_mix],
    out_specs=_spec_rows,
    out_shape=_rows_out,
)


def kernel(x, edge_index, W0, b0, W1, b1, prelu_a, mixing):
    row = edge_index[0]
    col = edge_index[1]
    b0r = b0.reshape(1, H)
    b1r = b1.reshape(1, H)
    ar = prelu_a.reshape(1, H)
    mixp = jnp.pad(mixing.astype(jnp.float32), ((0, 8 - T), (0, 0)),
                   constant_values=-1e30)

    col3 = col.reshape(NW, DNCH, DCH)
    w = jnp.arange(NW, dtype=jnp.int32)[:, None]
    prow = (w * 113 + jnp.arange(PAD, dtype=jnp.int32)[None, :] * 89) % N
    pcol = jnp.full((NW, PAD), N, dtype=jnp.int32) + w // NC
    rowp = jnp.concatenate([row.reshape(NW, EPW), prow], axis=1).reshape(-1)
    colp = jnp.concatenate([col.reshape(NW, EPW), pcol], axis=1).reshape(-1)

    degp = _degree_kernel(col3).reshape(NC, N, 1)

    u0, ut0, ut1, ut2 = _tc_a(x, W0, degp)

    cts = [jnp.full((1,), 0.5 ** t, dtype=jnp.float32) for t in range(T)]

    p10 = _spmv_kernel(ut0, rowp, colp)
    p11 = _spmv_kernel(ut1, rowp, colp)
    v0, u10 = _tc_b(cts[0], p10, u0, degp, W1, b0r, ar)
    p12 = _spmv_kernel(ut2, rowp, colp)
    v1, u11 = _tc_b(cts[1], p11, u0, degp, W1, b0r, ar)
    p20 = _spmv_kernel(v0, rowp, colp)
    v2, u12 = _tc_b(cts[2], p12, u0, degp, W1, b0r, ar)
    p21 = _spmv_kernel(v1, rowp, colp)
    p22 = _spmv_kernel(v2, rowp, colp)

    features = _tc_c(p20, p21, p22, u10, u11, u12, degp, b1r, ar, mixp)

    edge_weight_last = jnp.full((E,), 0.25, dtype=jnp.float32)
    return (features, edge_index, edge_weight_last)


# speedup vs baseline: 1.0387x; 1.0156x over previous
import functools

import jax
import jax.numpy as jnp
from jax import lax
from jax.experimental import pallas as pl
from jax.experimental.pallas import tpu as pltpu
from jax.experimental.pallas import tpu_sc as plsc

N = 10000
E = 320000
D = 128
H = 128
T = 3

NC = 2
NS = 16
NW = NC * NS
EPW = E // NW
CHUNK = 128
PAD = -EPW % CHUNK
EPWP = EPW + PAD
NCH = EPWP // CHUNK
DCH = 80
DNCH = EPW // DCH
RPT = N // NS
RPT8 = 624

_mesh = plsc.VectorSubcoreMesh(core_axis_name="c", subcore_axis_name="s")


def _worker(c, s):
    return s * NC + c


@functools.partial(
    pl.kernel,
    out_type=jax.ShapeDtypeStruct((NC * N,), jnp.float32),
    mesh=_mesh,
    scratch_types=[
        pltpu.VMEM((DNCH, DCH), jnp.int32),
        pltpu.VMEM((DCH,), jnp.float32),
        pltpu.VMEM((RPT8,), jnp.float32),
        pltpu.VMEM_SHARED((N,), jnp.float32),
        pltpu.SemaphoreType.DMA,
        pltpu.SemaphoreType.DMA,
    ],
)
def _degree_kernel(col3_hbm, out_hbm, col_v2, ones_v, zeros_v, acc_sh,
                   ssem0, ssem1):
    c = lax.axis_index("c")
    s = lax.axis_index("s")
    wid = _worker(c, s)
    ssems = (ssem0, ssem1)

    pltpu.sync_copy(col3_hbm.at[wid], col_v2)

    def fill(i, _):
        ones_v[pl.ds(i * 16, 16)] = jnp.ones((16,), jnp.float32)
        return _

    lax.fori_loop(0, DCH // 16, fill, 0)

    def zfill(i, _):
        zeros_v[pl.ds(i * 16, 16)] = jnp.zeros((16,), jnp.float32)
        return _

    lax.fori_loop(0, RPT8 // 16, zfill, 0)

    pltpu.sync_copy(zeros_v, acc_sh.at[pl.ds(s * RPT8, RPT8)])

    @pl.when(s == NS - 1)
    def _():
        pltpu.sync_copy(zeros_v.at[pl.ds(0, 16)],
                        acc_sh.at[pl.ds(NS * RPT8, N - NS * RPT8)])

    plsc.subcore_barrier()

    def s_start(k, j):
        pltpu.async_copy(ones_v, acc_sh.at[col_v2.at[k]], ssems[j], add=True)

    def s_wait(j):
        pltpu.make_async_copy(ones_v, acc_sh.at[col_v2.at[0]],
                              ssems[j]).wait()

    def stage(k, j):
        @pl.when(k >= 2)
        def _():
            s_wait(j)

        s_start(k, j)

    def pair(i, _):
        stage(2 * i, 0)
        stage(2 * i + 1, 1)
        return _

    lax.fori_loop(0, DNCH // 2, pair, 0)
    stage(DNCH - 1, 0)
    s_wait(1)
    s_wait(0)
    plsc.subcore_barrier()

    obase = c * N
    pltpu.sync_copy(acc_sh.at[pl.ds(s * RPT8, RPT8)], zeros_v)
    pltpu.sync_copy(zeros_v, out_hbm.at[pl.ds(obase + s * RPT8, RPT8)])

    @pl.when(s == NS - 1)
    def _():
        tail = N - NS * RPT8
        pltpu.sync_copy(acc_sh.at[pl.ds(NS * RPT8, tail)],
                        ones_v.at[pl.ds(0, tail)])
        pltpu.sync_copy(ones_v.at[pl.ds(0, tail)],
                        out_hbm.at[pl.ds(obase + NS * RPT8, tail)])


_part_out = jax.ShapeDtypeStruct((NC, N, D), jnp.float32)


@functools.partial(
    pl.kernel,
    out_type=_part_out,
    mesh=_mesh,
    scratch_types=[
        pltpu.VMEM((3, CHUNK), jnp.int32),
        pltpu.VMEM((3, CHUNK), jnp.int32),
        pltpu.VMEM((CHUNK, D), jnp.float32),
        pltpu.VMEM((CHUNK, D), jnp.float32),
        pltpu.VMEM((CHUNK, D), jnp.float32),
        pltpu.VMEM_SHARED((N + NS, D), jnp.float32),
        pltpu.SemaphoreType.DMA,
        pltpu.SemaphoreType.DMA,
        pltpu.SemaphoreType.DMA,
        pltpu.SemaphoreType.DMA,
        pltpu.SemaphoreType.DMA,
    ],
)
def _spmv_kernel(v_hbm, row_hbm, col_hbm, out_hbm,
                 row_i, col_i, buf0, buf1, buf2, acc_sh,
                 gsem0, gsem1, gsem2, ssem, isem):
    c = lax.axis_index("c")
    s = lax.axis_index("s")
    wid = _worker(c, s)
    bufs = (buf0, buf1, buf2)
    gsems = (gsem0, gsem1, gsem2)

    ebase = wid * EPWP

    def zrow(i, _):
        for j in range(D // 16):
            buf0[i, pl.ds(j * 16, 16)] = jnp.zeros((16,), jnp.float32)
        return _

    lax.fori_loop(0, CHUNK, zrow, 0)

    rbase = s * RPT8
    rem = RPT8 % CHUNK
    tail = N - NS * RPT8

    for i in range(RPT8 // CHUNK):
        pltpu.async_copy(buf0, acc_sh.at[pl.ds(rbase + i * CHUNK, CHUNK)],
                         ssem)
    pltpu.async_copy(buf0.at[pl.ds(0, rem)],
                     acc_sh.at[pl.ds(rbase + RPT8 - rem, rem)], ssem)

    @pl.when(s == NS - 1)
    def _():
        pltpu.async_copy(buf0.at[pl.ds(0, tail)],
                         acc_sh.at[pl.ds(NS * RPT8, tail)], ssem)

    for i in range(RPT8 // CHUNK):
        pltpu.make_async_copy(buf0, acc_sh.at[pl.ds(rbase, CHUNK)],
                              ssem).wait()
    pltpu.make_async_copy(buf0.at[pl.ds(0, rem)],
                          acc_sh.at[pl.ds(rbase, rem)], ssem).wait()

    @pl.when(s == NS - 1)
    def _():
        pltpu.make_async_copy(buf0.at[pl.ds(0, tail)],
                              acc_sh.at[pl.ds(rbase, tail)], ssem).wait()

    plsc.subcore_barrier()

    def i_start(k, m):
        off = ebase + k * CHUNK
        pltpu.async_copy(row_hbm.at[pl.ds(off, CHUNK)], row_i.at[m], isem)
        pltpu.async_copy(col_hbm.at[pl.ds(off, CHUNK)], col_i.at[m], isem)

    def i_wait(m):
        pltpu.make_async_copy(row_hbm.at[pl.ds(0, CHUNK)], row_i.at[m],
                              isem).wait()
        pltpu.make_async_copy(col_hbm.at[pl.ds(0, CHUNK)], col_i.at[m],
                              isem).wait()

    def g_start(k, j):
        pltpu.async_copy(v_hbm.at[row_i.at[j]], bufs[j], gsems[j])

    def g_wait(j):
        pltpu.make_async_copy(v_hbm.at[row_i.at[0]], bufs[j],
                              gsems[j]).wait()

    def s_start(j):
        pltpu.async_copy(bufs[j], acc_sh.at[col_i.at[j]], ssem, add=True)

    def s_wait(j):
        pltpu.make_async_copy(bufs[j], acc_sh.at[col_i.at[0]], ssem).wait()

    def stage(k, q):
        j = q % 3

        @pl.when(k >= 1)
        def _():
            s_wait((q - 1) % 3)

        @pl.when(k + 1 < NCH)
        def _():
            i_wait((q + 1) % 3)
            g_start(k + 1, (q + 1) % 3)

        g_wait(j)
        s_start(j)

        @pl.when(k + 2 < NCH)
        def _():
            i_start(k + 2, (q + 2) % 3)

    i_start(0, 0)
    i_wait(0)
    g_start(0, 0)
    i_start(1, 1)

    def triple(i, _):
        for q in range(3):
            stage(3 * i + q, q)
        return _

    lax.fori_loop(0, NCH // 3, triple, 0)
    for k in range(NCH - NCH % 3, NCH):
        stage(k, k)
    s_wait((NCH - 1) % 3)
    plsc.subcore_barrier()

    wb = [(rbase + i * CHUNK, CHUNK) for i in range(RPT8 // CHUNK)]
    wb.append((rbase + RPT8 - rem, rem))
    wsems = (gsem0, gsem1)
    for i, (off, sz) in enumerate(wb):
        j = i % 2
        if i >= 2:
            poff, psz = wb[i - 2]
            pltpu.make_async_copy(bufs[j].at[pl.ds(0, psz)],
                                  out_hbm.at[c, pl.ds(poff, psz)],
                                  wsems[j]).wait()
        pltpu.sync_copy(acc_sh.at[pl.ds(off, sz)], bufs[j].at[pl.ds(0, sz)])
        pltpu.async_copy(bufs[j].at[pl.ds(0, sz)],
                         out_hbm.at[c, pl.ds(off, sz)], wsems[j])
    for i in (len(wb) - 2, len(wb) - 1):
        poff, psz = wb[i]
        pltpu.make_async_copy(bufs[i % 2].at[pl.ds(0, psz)],
                              out_hbm.at[c, pl.ds(poff, psz)],
                              wsems[i % 2]).wait()

    @pl.when(s == NS - 1)
    def _():
        pltpu.sync_copy(acc_sh.at[pl.ds(NS * RPT8, tail)],
                        buf2.at[pl.ds(0, tail)])
        pltpu.sync_copy(buf2.at[pl.ds(0, tail)],
                        out_hbm.at[c, pl.ds(NS * RPT8, tail)])


_BR = 1000
_GRID = N // _BR


def _tc_a_body(x_ref, w0_ref, degp_ref, u0_ref, ut0_ref, ut1_ref, ut2_ref):
    u = jnp.dot(x_ref[...], w0_ref[...], preferred_element_type=jnp.float32)
    u0_ref[...] = u
    deg = degp_ref[0] + degp_ref[1]
    ut_refs = (ut0_ref, ut1_ref, ut2_ref)
    for t in range(T):
        ct = 0.5 ** t
        d = lax.rsqrt(ct * deg + 1.0)
        ut_refs[t][...] = d * u


def _tc_b_body(ct_ref, p_ref, u0_ref, degp_ref, w1_ref, b0_ref,
               a_ref, v_ref, u1_ref):
    u0 = u0_ref[...]
    deg = degp_ref[0] + degp_ref[1]
    ct = ct_ref[0]
    d = lax.rsqrt(ct * deg + 1.0)
    st = p_ref[0] + p_ref[1]
    z = d * (ct * st) + (d * d) * u0 + b0_ref[...]
    z = jnp.where(z >= 0, z, a_ref[...] * z)
    u1 = jnp.dot(z, w1_ref[...], preferred_element_type=jnp.float32)
    u1_ref[...] = u1
    v_ref[...] = d * u1


def _tc_c_body(p0_ref, p1_ref, p2_ref, u10_ref, u11_ref, u12_ref, degp_ref,
               b1_ref, a_ref, mix_ref, out_ref):
    deg = degp_ref[0] + degp_ref[1]
    b1 = b1_ref[...]
    a = a_ref[...]
    m = mix_ref[...]
    e = jnp.exp(m - jnp.max(m, axis=0, keepdims=True))
    coeff = e / jnp.sum(e, axis=0, keepdims=True)
    p_refs = (p0_ref, p1_ref, p2_ref)
    u1_refs = (u10_ref, u11_ref, u12_ref)
    acc = jnp.zeros((_BR, D), jnp.float32)
    for t in range(T):
        ct = 0.5 ** t
        d = lax.rsqrt(ct * deg + 1.0)
        st = p_refs[t][0] + p_refs[t][1]
        u1 = u1_refs[t][...]
        z = d * (ct * st) + (d * d) * u1 + b1
        z = jnp.where(z >= 0, z, a * z)
        acc = acc + coeff[t:t + 1, 0:1] * z
    out_ref[...] = acc


def _row_block(i):
    return (i, 0)


_spec_rows = pl.BlockSpec((_BR, D), _row_block)
_spec_full = pl.BlockSpec((D, D), lambda i: (0, 0))
_spec_vec = pl.BlockSpec((1, D), lambda i: (0, 0))
_spec_deg = pl.BlockSpec((NC, _BR, 1), lambda i: (0, i, 0))
_spec_part = pl.BlockSpec((NC, _BR, D), lambda i: (0, i, 0))
_spec_mix = pl.BlockSpec((8, 1), lambda i: (0, 0))

_rows_out = jax.ShapeDtypeStruct((N, D), jnp.float32)

_tc_a = pl.pallas_call(
    _tc_a_body,
    grid=(_GRID,),
    in_specs=[_spec_rows, _spec_full, _spec_deg],
    out_specs=[_spec_rows] * 4,
    out_shape=[_rows_out] * 4,
)

_spec_ct = pl.BlockSpec(memory_space=pltpu.SMEM)

_tc_b = pl.pallas_call(
    _tc_b_body,
    grid=(_GRID,),
    in_specs=[_spec_ct, _spec_part, _spec_rows, _spec_deg,
              _spec_full, _spec_vec, _spec_vec],
    out_specs=[_spec_rows] * 2,
    out_shape=[_rows_out] * 2,
)

_tc_c = pl.pallas_call(
    _tc_c_body,
    grid=(_GRID,),
    in_specs=[_spec_part, _spec_part, _spec_part, _spec_rows, _spec_rows,
              _spec_rows, _spec_deg, _spec_vec, _spec_vec, _spec_mix],
    out_specs=_spec_rows,
    out_shape=_rows_out,
)


def kernel(x, edge_index, W0, b0, W1, b1, prelu_a, mixing):
    row = edge_index[0]
    col = edge_index[1]
    b0r = b0.reshape(1, H)
    b1r = b1.reshape(1, H)
    ar = prelu_a.reshape(1, H)
    mixp = jnp.pad(mixing.astype(jnp.float32), ((0, 8 - T), (0, 0)),
                   constant_values=-1e30)

    col3 = col.reshape(NW, DNCH, DCH)
    w = jnp.arange(NW, dtype=jnp.int32)[:, None]
    prow = (w * 113 + jnp.arange(PAD, dtype=jnp.int32)[None, :] * 89) % N
    pcol = jnp.full((NW, PAD), N, dtype=jnp.int32) + w // NC
    rowp = jnp.concatenate([row.reshape(NW, EPW), prow], axis=1).reshape(-1)
    colp = jnp.concatenate([col.reshape(NW, EPW), pcol], axis=1).reshape(-1)

    degp = _degree_kernel(col3).reshape(NC, N, 1)

    u0, ut0, ut1, ut2 = _tc_a(x, W0, degp)

    cts = [jnp.full((1,), 0.5 ** t, dtype=jnp.float32) for t in range(T)]

    p10 = _spmv_kernel(ut0, rowp, colp)
    p11 = _spmv_kernel(ut1, rowp, colp)
    v0, u10 = _tc_b(cts[0], p10, u0, degp, W1, b0r, ar)
    p12 = _spmv_kernel(ut2, rowp, colp)
    v1, u11 = _tc_b(cts[1], p11, u0, degp, W1, b0r, ar)
    p20 = _spmv_kernel(v0, rowp, colp)
    v2, u12 = _tc_b(cts[2], p12, u0, degp, W1, b0r, ar)
    p21 = _spmv_kernel(v1, rowp, colp)
    p22 = _spmv_kernel(v2, rowp, colp)

    features = _tc_c(p20, p21, p22, u10, u11, u12, degp, b1r, ar, mixp)

    edge_weight_last = jnp.full((E,), 0.25, dtype=jnp.float32)
    return (features, edge_index, edge_weight_last)


# speedup vs baseline: 1.0446x; 1.0057x over previous
import functools

import jax
import jax.numpy as jnp
from jax import lax
from jax.experimental import pallas as pl
from jax.experimental.pallas import tpu as pltpu
from jax.experimental.pallas import tpu_sc as plsc

N = 10000
E = 320000
D = 128
H = 128
T = 3

NC = 2
NS = 16
NW = NC * NS
EPW = E // NW
CHUNK = 128
PAD = -EPW % CHUNK
EPWP = EPW + PAD
NCH = EPWP // CHUNK
DCH = 80
DNCH = EPW // DCH
RPT = N // NS
RPT8 = 624

_mesh = plsc.VectorSubcoreMesh(core_axis_name="c", subcore_axis_name="s")


def _worker(c, s):
    return s * NC + c


@functools.partial(
    pl.kernel,
    out_type=jax.ShapeDtypeStruct((NC * N,), jnp.float32),
    mesh=_mesh,
    scratch_types=[
        pltpu.VMEM((DNCH, DCH), jnp.int32),
        pltpu.VMEM((DCH,), jnp.float32),
        pltpu.VMEM((RPT8,), jnp.float32),
        pltpu.VMEM_SHARED((N,), jnp.float32),
        pltpu.SemaphoreType.DMA,
        pltpu.SemaphoreType.DMA,
    ],
)
def _degree_kernel(col3_hbm, out_hbm, col_v2, ones_v, zeros_v, acc_sh,
                   ssem0, ssem1):
    c = lax.axis_index("c")
    s = lax.axis_index("s")
    wid = _worker(c, s)
    ssems = (ssem0, ssem1)

    pltpu.sync_copy(col3_hbm.at[wid], col_v2)

    def fill(i, _):
        ones_v[pl.ds(i * 16, 16)] = jnp.ones((16,), jnp.float32)
        return _

    lax.fori_loop(0, DCH // 16, fill, 0)

    def zfill(i, _):
        zeros_v[pl.ds(i * 16, 16)] = jnp.zeros((16,), jnp.float32)
        return _

    lax.fori_loop(0, RPT8 // 16, zfill, 0)

    pltpu.sync_copy(zeros_v, acc_sh.at[pl.ds(s * RPT8, RPT8)])

    @pl.when(s == NS - 1)
    def _():
        pltpu.sync_copy(zeros_v.at[pl.ds(0, 16)],
                        acc_sh.at[pl.ds(NS * RPT8, N - NS * RPT8)])

    plsc.subcore_barrier()

    def s_start(k, j):
        pltpu.async_copy(ones_v, acc_sh.at[col_v2.at[k]], ssems[j], add=True)

    def s_wait(j):
        pltpu.make_async_copy(ones_v, acc_sh.at[col_v2.at[0]],
                              ssems[j]).wait()

    def stage(k, j):
        @pl.when(k >= 2)
        def _():
            s_wait(j)

        s_start(k, j)

    def pair(i, _):
        stage(2 * i, 0)
        stage(2 * i + 1, 1)
        return _

    lax.fori_loop(0, DNCH // 2, pair, 0)
    stage(DNCH - 1, 0)
    s_wait(1)
    s_wait(0)
    plsc.subcore_barrier()

    obase = c * N
    pltpu.sync_copy(acc_sh.at[pl.ds(s * RPT8, RPT8)], zeros_v)
    pltpu.sync_copy(zeros_v, out_hbm.at[pl.ds(obase + s * RPT8, RPT8)])

    @pl.when(s == NS - 1)
    def _():
        tail = N - NS * RPT8
        pltpu.sync_copy(acc_sh.at[pl.ds(NS * RPT8, tail)],
                        ones_v.at[pl.ds(0, tail)])
        pltpu.sync_copy(ones_v.at[pl.ds(0, tail)],
                        out_hbm.at[pl.ds(obase + NS * RPT8, tail)])


_part_out = jax.ShapeDtypeStruct((NC, N, D), jnp.float32)


@functools.partial(
    pl.kernel,
    out_type=_part_out,
    mesh=_mesh,
    scratch_types=[
        pltpu.VMEM((3, CHUNK), jnp.int32),
        pltpu.VMEM((3, CHUNK), jnp.int32),
        pltpu.VMEM((CHUNK, D), jnp.float32),
        pltpu.VMEM((CHUNK, D), jnp.float32),
        pltpu.VMEM((CHUNK, D), jnp.float32),
        pltpu.VMEM_SHARED((N + NS, D), jnp.float32),
        pltpu.SemaphoreType.DMA,
        pltpu.SemaphoreType.DMA,
        pltpu.SemaphoreType.DMA,
        pltpu.SemaphoreType.DMA,
        pltpu.SemaphoreType.DMA,
    ],
)
def _spmv_kernel(v_hbm, row_hbm, col_hbm, out_hbm,
                 row_i, col_i, buf0, buf1, buf2, acc_sh,
                 gsem0, gsem1, gsem2, ssem, isem):
    c = lax.axis_index("c")
    s = lax.axis_index("s")
    wid = _worker(c, s)
    bufs = (buf0, buf1, buf2)
    gsems = (gsem0, gsem1, gsem2)

    ebase = wid * EPWP

    def zrow(i, _):
        for j in range(D // 16):
            buf0[i, pl.ds(j * 16, 16)] = jnp.zeros((16,), jnp.float32)
        return _

    lax.fori_loop(0, CHUNK, zrow, 0)

    rbase = s * RPT8
    rem = RPT8 % CHUNK
    tail = N - NS * RPT8

    for i in range(RPT8 // CHUNK):
        pltpu.async_copy(buf0, acc_sh.at[pl.ds(rbase + i * CHUNK, CHUNK)],
                         ssem)
    pltpu.async_copy(buf0.at[pl.ds(0, rem)],
                     acc_sh.at[pl.ds(rbase + RPT8 - rem, rem)], ssem)

    @pl.when(s == NS - 1)
    def _():
        pltpu.async_copy(buf0.at[pl.ds(0, tail)],
                         acc_sh.at[pl.ds(NS * RPT8, tail)], ssem)

    for i in range(RPT8 // CHUNK):
        pltpu.make_async_copy(buf0, acc_sh.at[pl.ds(rbase, CHUNK)],
                              ssem).wait()
    pltpu.make_async_copy(buf0.at[pl.ds(0, rem)],
                          acc_sh.at[pl.ds(rbase, rem)], ssem).wait()

    @pl.when(s == NS - 1)
    def _():
        pltpu.make_async_copy(buf0.at[pl.ds(0, tail)],
                              acc_sh.at[pl.ds(rbase, tail)], ssem).wait()

    plsc.subcore_barrier()

    def i_start(k, m):
        off = ebase + k * CHUNK
        pltpu.async_copy(row_hbm.at[pl.ds(off, CHUNK)], row_i.at[m], isem)
        pltpu.async_copy(col_hbm.at[pl.ds(off, CHUNK)], col_i.at[m], isem)

    def i_wait(m):
        pltpu.make_async_copy(row_hbm.at[pl.ds(0, CHUNK)], row_i.at[m],
                              isem).wait()
        pltpu.make_async_copy(col_hbm.at[pl.ds(0, CHUNK)], col_i.at[m],
                              isem).wait()

    def g_start(k, j):
        pltpu.async_copy(v_hbm.at[row_i.at[j]], bufs[j], gsems[j])

    def g_wait(j):
        pltpu.make_async_copy(v_hbm.at[row_i.at[0]], bufs[j],
                              gsems[j]).wait()

    def s_start(j):
        pltpu.async_copy(bufs[j], acc_sh.at[col_i.at[j]], ssem, add=True)

    def s_wait(j):
        pltpu.make_async_copy(bufs[j], acc_sh.at[col_i.at[0]], ssem).wait()

    def stage(k, q):
        j = q % 3

        @pl.when(k + 1 < NCH)
        def _():
            i_wait((q + 1) % 3)
            g_start(k + 1, (q + 1) % 3)

        @pl.when(k >= 1)
        def _():
            s_wait((q - 1) % 3)

        g_wait(j)
        s_start(j)

        @pl.when(k + 2 < NCH)
        def _():
            i_start(k + 2, (q + 2) % 3)

    i_start(0, 0)
    i_wait(0)
    g_start(0, 0)
    i_start(1, 1)

    def triple(i, _):
        for q in range(3):
            stage(3 * i + q, q)
        return _

    lax.fori_loop(0, NCH // 3, triple, 0)
    for k in range(NCH - NCH % 3, NCH):
        stage(k, k)
    s_wait((NCH - 1) % 3)
    plsc.subcore_barrier()

    wb = [(rbase + i * CHUNK, CHUNK) for i in range(RPT8 // CHUNK)]
    wb.append((rbase + RPT8 - rem, rem))
    wsems = (gsem0, gsem1)
    for i, (off, sz) in enumerate(wb):
        j = i % 2
        if i >= 2:
            poff, psz = wb[i - 2]
            pltpu.make_async_copy(bufs[j].at[pl.ds(0, psz)],
                                  out_hbm.at[c, pl.ds(poff, psz)],
                                  wsems[j]).wait()
        pltpu.sync_copy(acc_sh.at[pl.ds(off, sz)], bufs[j].at[pl.ds(0, sz)])
        pltpu.async_copy(bufs[j].at[pl.ds(0, sz)],
                         out_hbm.at[c, pl.ds(off, sz)], wsems[j])
    for i in (len(wb) - 2, len(wb) - 1):
        poff, psz = wb[i]
        pltpu.make_async_copy(bufs[i % 2].at[pl.ds(0, psz)],
                              out_hbm.at[c, pl.ds(poff, psz)],
                              wsems[i % 2]).wait()

    @pl.when(s == NS - 1)
    def _():
        pltpu.sync_copy(acc_sh.at[pl.ds(NS * RPT8, tail)],
                        buf2.at[pl.ds(0, tail)])
        pltpu.sync_copy(buf2.at[pl.ds(0, tail)],
                        out_hbm.at[c, pl.ds(NS * RPT8, tail)])


_BR = 1000
_GRID = N // _BR


def _tc_a_body(x_ref, w0_ref, degp_ref, u0_ref, ut0_ref, ut1_ref, ut2_ref):
    u = jnp.dot(x_ref[...], w0_ref[...], preferred_element_type=jnp.float32)
    u0_ref[...] = u
    deg = degp_ref[0] + degp_ref[1]
    ut_refs = (ut0_ref, ut1_ref, ut2_ref)
    for t in range(T):
        ct = 0.5 ** t
        d = lax.rsqrt(ct * deg + 1.0)
        ut_refs[t][...] = d * u


def _tc_b_body(ct_ref, p_ref, u0_ref, degp_ref, w1_ref, b0_ref,
               a_ref, v_ref, u1_ref):
    u0 = u0_ref[...]
    deg = degp_ref[0] + degp_ref[1]
    ct = ct_ref[0]
    d = lax.rsqrt(ct * deg + 1.0)
    st = p_ref[0] + p_ref[1]
    z = d * (ct * st) + (d * d) * u0 + b0_ref[...]
    z = jnp.where(z >= 0, z, a_ref[...] * z)
    u1 = jnp.dot(z, w1_ref[...], preferred_element_type=jnp.float32)
    u1_ref[...] = u1
    v_ref[...] = d * u1


def _tc_c_body(p0_ref, p1_ref, p2_ref, u10_ref, u11_ref, u12_ref, degp_ref,
               b1_ref, a_ref, mix_ref, out_ref):
    deg = degp_ref[0] + degp_ref[1]
    b1 = b1_ref[...]
    a = a_ref[...]
    m = mix_ref[...]
    e = jnp.exp(m - jnp.max(m, axis=0, keepdims=True))
    coeff = e / jnp.sum(e, axis=0, keepdims=True)
    p_refs = (p0_ref, p1_ref, p2_ref)
    u1_refs = (u10_ref, u11_ref, u12_ref)
    acc = jnp.zeros((_BR, D), jnp.float32)
    for t in range(T):
        ct = 0.5 ** t
        d = lax.rsqrt(ct * deg + 1.0)
        st = p_refs[t][0] + p_refs[t][1]
        u1 = u1_refs[t][...]
        z = d * (ct * st) + (d * d) * u1 + b1
        z = jnp.where(z >= 0, z, a * z)
        acc = acc + coeff[t:t + 1, 0:1] * z
    out_ref[...] = acc


def _row_block(i):
    return (i, 0)


_spec_rows = pl.BlockSpec((_BR, D), _row_block)
_spec_full = pl.BlockSpec((D, D), lambda i: (0, 0))
_spec_vec = pl.BlockSpec((1, D), lambda i: (0, 0))
_spec_deg = pl.BlockSpec((NC, _BR, 1), lambda i: (0, i, 0))
_spec_part = pl.BlockSpec((NC, _BR, D), lambda i: (0, i, 0))
_spec_mix = pl.BlockSpec((8, 1), lambda i: (0, 0))

_rows_out = jax.ShapeDtypeStruct((N, D), jnp.float32)

_tc_a = pl.pallas_call(
    _tc_a_body,
    grid=(_GRID,),
    in_specs=[_spec_rows, _spec_full, _spec_deg],
    out_specs=[_spec_rows] * 4,
    out_shape=[_rows_out] * 4,
)

_spec_ct = pl.BlockSpec(memory_space=pltpu.SMEM)

_tc_b = pl.pallas_call(
    _tc_b_body,
    grid=(_GRID,),
    in_specs=[_spec_ct, _spec_part, _spec_rows, _spec_deg,
              _spec_full, _spec_vec, _spec_vec],
    out_specs=[_spec_rows] * 2,
    out_shape=[_rows_out] * 2,
)

_tc_c = pl.pallas_call(
    _tc_c_body,
    grid=(_GRID,),
    in_specs=[_spec_part, _spec_part, _spec_part, _spec_rows, _spec_rows,
              _spec_rows, _spec_deg, _spec_vec, _spec_vec, _spec_mix],
    out_specs=_spec_rows,
    out_shape=_rows_out,
)


def kernel(x, edge_index, W0, b0, W1, b1, prelu_a, mixing):
    row = edge_index[0]
    col = edge_index[1]
    b0r = b0.reshape(1, H)
    b1r = b1.reshape(1, H)
    ar = prelu_a.reshape(1, H)
    mixp = jnp.pad(mixing.astype(jnp.float32), ((0, 8 - T), (0, 0)),
                   constant_values=-1e30)

    col3 = col.reshape(NW, DNCH, DCH)
    w = jnp.arange(NW, dtype=jnp.int32)[:, None]
    prow = (w * 113 + jnp.arange(PAD, dtype=jnp.int32)[None, :] * 89) % N
    pcol = jnp.full((NW, PAD), N, dtype=jnp.int32) + w // NC
    rowp = jnp.concatenate([row.reshape(NW, EPW), prow], axis=1).reshape(-1)
    colp = jnp.concatenate([col.reshape(NW, EPW), pcol], axis=1).reshape(-1)

    degp = _degree_kernel(col3).reshape(NC, N, 1)

    u0, ut0, ut1, ut2 = _tc_a(x, W0, degp)

    cts = [jnp.full((1,), 0.5 ** t, dtype=jnp.float32) for t in range(T)]

    p10 = _spmv_kernel(ut0, rowp, colp)
    p11 = _spmv_kernel(ut1, rowp, colp)
    v0, u10 = _tc_b(cts[0], p10, u0, degp, W1, b0r, ar)
    p12 = _spmv_kernel(ut2, rowp, colp)
    v1, u11 = _tc_b(cts[1], p11, u0, degp, W1, b0r, ar)
    p20 = _spmv_kernel(v0, rowp, colp)
    v2, u12 = _tc_b(cts[2], p12, u0, degp, W1, b0r, ar)
    p21 = _spmv_kernel(v1, rowp, colp)
    p22 = _spmv_kernel(v2, rowp, colp)

    features = _tc_c(p20, p21, p22, u10, u11, u12, degp, b1r, ar, mixp)

    edge_weight_last = jnp.full((E,), 0.25, dtype=jnp.float32)
    return (features, edge_index, edge_weight_last)
